# baseline (speedup 1.0000x reference)
import functools

import jax
import jax.numpy as jnp
from jax import lax
from jax.experimental import pallas as pl
from jax.experimental.pallas import tpu as pltpu
from jax.experimental.pallas import tpu_sc as plsc

_K = 16
_B = 4
_D = 3
_N = 4096
_QB = 128
_NW = 32
_CHUNK = _B * _N // _NW
_TPB = _NW // _B


def _topk_body(pcq_ref, pcr_ref, idx_ref, vals_ref):
    qblk = pl.program_id(1)
    q = pcq_ref[0]
    r = pcr_ref[0]
    sq_q = jnp.sum(q * q, axis=0)
    sq_r = jnp.sum(r * r, axis=0)
    qb = q.astype(jnp.bfloat16).astype(jnp.float32)
    rb = r.astype(jnp.bfloat16).astype(jnp.float32)
    inner = (qb[0][:, None] * rb[0][None, :]
             + qb[1][:, None] * rb[1][None, :]
             + qb[2][:, None] * rb[2][None, :])
    d2 = (sq_r[None, :] + sq_q[:, None]) - 2.0 * inner

    lane = lax.broadcasted_iota(jnp.int32, (_QB, _N), 1)
    inf = jnp.float32(jnp.inf)
    vals_ref[...] = d2

    krow = lax.broadcasted_iota(jnp.int32, (_K + 1, _QB), 0)

    def round_body(kk, state):
        j_prev, j_all = state
        vals = jnp.where(lane == j_prev[:, None], inf, vals_ref[...])
        vals_ref[...] = vals
        j = jnp.argmin(vals, axis=1).astype(jnp.int32)
        j_all = jnp.where(krow == kk, j[None, :], j_all)
        return j, j_all

    _, j_all = lax.fori_loop(
        0, _K + 1, round_body,
        (jnp.full((_QB,), -1, jnp.int32),
         jnp.zeros((_K + 1, _QB), jnp.int32)))
    idx_ref[0] = j_all[1:, :]


def _topk(point_cloud):
    return pl.pallas_call(
        _topk_body,
        grid=(_B, _N // _QB),
        in_specs=[
            pl.BlockSpec((1, _D, _QB), lambda b, q: (b, 0, q)),
            pl.BlockSpec((1, _D, _N), lambda b, q: (b, 0, 0)),
        ],
        out_specs=pl.BlockSpec((1, _K, _QB), lambda b, q: (b, 0, q)),
        out_shape=jax.ShapeDtypeStruct((_B, _K, _N), jnp.int32),
        scratch_shapes=[pltpu.VMEM((_QB, _N), jnp.float32)],
        compiler_params=pltpu.CompilerParams(
            dimension_semantics=("parallel", "parallel")),
    )(point_cloud, point_cloud)


def _edge_body(pc_hbm, idx_hbm, out_hbm, pcx_v, pcy_v, pcz_v, idx_v, out_v):
    wid = lax.axis_index("s") * 2 + lax.axis_index("c")
    b = wid // _TPB
    t = wid % _TPB
    n0 = pl.multiple_of(t * _CHUNK, _CHUNK)
    pltpu.sync_copy(pc_hbm.at[pl.ds((b * 3 + 0) * _N, _N)], pcx_v)
    pltpu.sync_copy(pc_hbm.at[pl.ds((b * 3 + 1) * _N, _N)], pcy_v)
    pltpu.sync_copy(pc_hbm.at[pl.ds((b * 3 + 2) * _N, _N)], pcz_v)
    for kk in range(_K):
        pltpu.sync_copy(idx_hbm.at[pl.ds((b * _K + kk) * _N + n0, _CHUNK)],
                        idx_v.at[pl.ds(kk * _CHUNK, _CHUNK)])

    def body(v, carry):
        off = pl.multiple_of(v * 16, 16)
        qoff = pl.multiple_of(n0 + off, 16)
        qx = pcx_v[pl.ds(qoff, 16)]
        qy = pcy_v[pl.ds(qoff, 16)]
        qz = pcz_v[pl.ds(qoff, 16)]
        for kk in range(_K):
            ii = idx_v[pl.ds(kk * _CHUNK + off, 16)]
            gx = plsc.load_gather(pcx_v, [ii])
            gy = plsc.load_gather(pcy_v, [ii])
            gz = plsc.load_gather(pcz_v, [ii])
            out_v[pl.ds((0 * _K + kk) * _CHUNK + off, 16)] = qx
            out_v[pl.ds((1 * _K + kk) * _CHUNK + off, 16)] = qy
            out_v[pl.ds((2 * _K + kk) * _CHUNK + off, 16)] = qz
            out_v[pl.ds((3 * _K + kk) * _CHUNK + off, 16)] = gx - qx
            out_v[pl.ds((4 * _K + kk) * _CHUNK + off, 16)] = gy - qy
            out_v[pl.ds((5 * _K + kk) * _CHUNK + off, 16)] = gz - qz
        return carry

    lax.fori_loop(0, _CHUNK // 16, body, 0)
    for c in range(2 * _D):
        for kk in range(_K):
            pltpu.sync_copy(
                out_v.at[pl.ds((c * _K + kk) * _CHUNK, _CHUNK)],
                out_hbm.at[pl.ds(((b * 2 * _D + c) * _K + kk) * _N + n0,
                                 _CHUNK)])


def _edge(point_cloud, idx):
    mesh = plsc.VectorSubcoreMesh(core_axis_name="c", subcore_axis_name="s")
    k = functools.partial(
        pl.kernel,
        mesh=mesh,
        out_type=jax.ShapeDtypeStruct((_B * 2 * _D * _K * _N,), jnp.float32),
        scratch_types=[
            pltpu.VMEM((_N,), jnp.float32),
            pltpu.VMEM((_N,), jnp.float32),
            pltpu.VMEM((_N,), jnp.float32),
            pltpu.VMEM((_K * _CHUNK,), jnp.int32),
            pltpu.VMEM((2 * _D * _K * _CHUNK,), jnp.float32),
        ],
        compiler_params=pltpu.CompilerParams(use_tc_tiling_on_sc=False,
                                             needs_layout_passes=False),
    )(_edge_body)
    out = k(point_cloud.reshape(-1), idx.reshape(-1))
    return out.reshape(_B, 2 * _D, _K, _N)


def kernel(point_cloud):
    idx = _topk(point_cloud)
    edge_feature = _edge(point_cloud, idx)
    return edge_feature, idx


# speedup vs baseline: 1.0248x; 1.0248x over previous
import functools

import jax
import jax.numpy as jnp
from jax import lax
from jax.experimental import pallas as pl
from jax.experimental.pallas import tpu as pltpu
from jax.experimental.pallas import tpu_sc as plsc

_K = 16
_B = 4
_D = 3
_N = 4096
_QB = 128
_NW = 32
_CHUNK = _B * _N // _NW
_TPB = _NW // _B


def _topk_body(pcq_ref, pcr_ref, idx_ref, vals_ref):
    qblk = pl.program_id(1)
    q = pcq_ref[0]
    r = pcr_ref[0]
    sq_q = jnp.sum(q * q, axis=0)
    sq_r = jnp.sum(r * r, axis=0)
    qb = q.astype(jnp.bfloat16).astype(jnp.float32)
    rb = r.astype(jnp.bfloat16).astype(jnp.float32)
    inner = (qb[0][:, None] * rb[0][None, :]
             + qb[1][:, None] * rb[1][None, :]
             + qb[2][:, None] * rb[2][None, :])
    d2 = (sq_r[None, :] + sq_q[:, None]) - 2.0 * inner

    lane = lax.broadcasted_iota(jnp.int32, (_QB, _N), 1)
    inf = jnp.float32(jnp.inf)
    vals_ref[...] = d2

    krow = lax.broadcasted_iota(jnp.int32, (_K + 1, _QB), 0)

    def round_body(kk, state):
        j_prev, j_all = state
        vals = jnp.where(lane == j_prev[:, None], inf, vals_ref[...])
        vals_ref[...] = vals
        m = jnp.min(vals, axis=1)
        cand = jnp.where(vals == m[:, None], lane, _N)
        j = jnp.min(cand, axis=1)
        j_all = jnp.where(krow == kk, j[None, :], j_all)
        return j, j_all

    _, j_all = lax.fori_loop(
        0, _K + 1, round_body,
        (jnp.full((_QB,), -1, jnp.int32),
         jnp.zeros((_K + 1, _QB), jnp.int32)))
    idx_ref[0] = j_all[1:, :]


def _topk(point_cloud):
    return pl.pallas_call(
        _topk_body,
        grid=(_B, _N // _QB),
        in_specs=[
            pl.BlockSpec((1, _D, _QB), lambda b, q: (b, 0, q)),
            pl.BlockSpec((1, _D, _N), lambda b, q: (b, 0, 0)),
        ],
        out_specs=pl.BlockSpec((1, _K, _QB), lambda b, q: (b, 0, q)),
        out_shape=jax.ShapeDtypeStruct((_B, _K, _N), jnp.int32),
        scratch_shapes=[pltpu.VMEM((_QB, _N), jnp.float32)],
        compiler_params=pltpu.CompilerParams(
            dimension_semantics=("parallel", "parallel")),
    )(point_cloud, point_cloud)


def _edge_body(pc_hbm, idx_hbm, out_hbm, pcx_v, pcy_v, pcz_v, idx_v, out_v):
    wid = lax.axis_index("s") * 2 + lax.axis_index("c")
    b = wid // _TPB
    t = wid % _TPB
    n0 = pl.multiple_of(t * _CHUNK, _CHUNK)
    pltpu.sync_copy(pc_hbm.at[pl.ds((b * 3 + 0) * _N, _N)], pcx_v)
    pltpu.sync_copy(pc_hbm.at[pl.ds((b * 3 + 1) * _N, _N)], pcy_v)
    pltpu.sync_copy(pc_hbm.at[pl.ds((b * 3 + 2) * _N, _N)], pcz_v)
    for kk in range(_K):
        pltpu.sync_copy(idx_hbm.at[pl.ds((b * _K + kk) * _N + n0, _CHUNK)],
                        idx_v.at[pl.ds(kk * _CHUNK, _CHUNK)])

    def body(v, carry):
        off = pl.multiple_of(v * 16, 16)
        qoff = pl.multiple_of(n0 + off, 16)
        qx = pcx_v[pl.ds(qoff, 16)]
        qy = pcy_v[pl.ds(qoff, 16)]
        qz = pcz_v[pl.ds(qoff, 16)]
        for kk in range(_K):
            ii = idx_v[pl.ds(kk * _CHUNK + off, 16)]
            gx = plsc.load_gather(pcx_v, [ii])
            gy = plsc.load_gather(pcy_v, [ii])
            gz = plsc.load_gather(pcz_v, [ii])
            out_v[pl.ds((0 * _K + kk) * _CHUNK + off, 16)] = qx
            out_v[pl.ds((1 * _K + kk) * _CHUNK + off, 16)] = qy
            out_v[pl.ds((2 * _K + kk) * _CHUNK + off, 16)] = qz
            out_v[pl.ds((3 * _K + kk) * _CHUNK + off, 16)] = gx - qx
            out_v[pl.ds((4 * _K + kk) * _CHUNK + off, 16)] = gy - qy
            out_v[pl.ds((5 * _K + kk) * _CHUNK + off, 16)] = gz - qz
        return carry

    lax.fori_loop(0, _CHUNK // 16, body, 0)
    for c in range(2 * _D):
        for kk in range(_K):
            pltpu.sync_copy(
                out_v.at[pl.ds((c * _K + kk) * _CHUNK, _CHUNK)],
                out_hbm.at[pl.ds(((b * 2 * _D + c) * _K + kk) * _N + n0,
                                 _CHUNK)])


def _edge(point_cloud, idx):
    mesh = plsc.VectorSubcoreMesh(core_axis_name="c", subcore_axis_name="s")
    k = functools.partial(
        pl.kernel,
        mesh=mesh,
        out_type=jax.ShapeDtypeStruct((_B * 2 * _D * _K * _N,), jnp.float32),
        scratch_types=[
            pltpu.VMEM((_N,), jnp.float32),
            pltpu.VMEM((_N,), jnp.float32),
            pltpu.VMEM((_N,), jnp.float32),
            pltpu.VMEM((_K * _CHUNK,), jnp.int32),
            pltpu.VMEM((2 * _D * _K * _CHUNK,), jnp.float32),
        ],
        compiler_params=pltpu.CompilerParams(use_tc_tiling_on_sc=False,
                                             needs_layout_passes=False),
    )(_edge_body)
    out = k(point_cloud.reshape(-1), idx.reshape(-1))
    return out.reshape(_B, 2 * _D, _K, _N)


def kernel(point_cloud):
    idx = _topk(point_cloud)
    edge_feature = _edge(point_cloud, idx)
    return edge_feature, idx


# speedup vs baseline: 1.3398x; 1.3073x over previous
import functools

import jax
import jax.numpy as jnp
from jax import lax
from jax.experimental import pallas as pl
from jax.experimental.pallas import tpu as pltpu
from jax.experimental.pallas import tpu_sc as plsc

_K = 16
_B = 4
_D = 3
_N = 4096
_QB = 256
_NW = 32
_CHUNK = _B * _N // _NW
_TPB = _NW // _B


def _topk_body(pcq_ref, pcr_ref, idx_ref, vals_ref):
    qblk = pl.program_id(1)
    q = pcq_ref[0]
    r = pcr_ref[0]
    sq_q = jnp.sum(q * q, axis=0)
    sq_r = jnp.sum(r * r, axis=0)
    qb = q.astype(jnp.bfloat16).astype(jnp.float32)
    rb = r.astype(jnp.bfloat16).astype(jnp.float32)
    inner = (qb[0][:, None] * rb[0][None, :]
             + qb[1][:, None] * rb[1][None, :]
             + qb[2][:, None] * rb[2][None, :])
    d2 = (sq_r[None, :] + sq_q[:, None]) - 2.0 * inner

    lane = lax.broadcasted_iota(jnp.int32, (_QB, _N), 1)
    inf = jnp.float32(jnp.inf)
    vals_ref[...] = d2

    krow = lax.broadcasted_iota(jnp.int32, (_K + 1, _QB), 0)

    def round_body(kk, j_all):
        vals = vals_ref[...]
        m = jnp.min(vals, axis=1)
        cand = jnp.where(vals == m[:, None], lane, _N)
        j = jnp.min(cand, axis=1)
        j_all = jnp.where(krow == kk, j[None, :], j_all)
        vals_ref[...] = jnp.where(lane == j[:, None], inf, vals)
        return j_all

    j_all = lax.fori_loop(0, _K + 1, round_body,
                          jnp.zeros((_K + 1, _QB), jnp.int32))
    idx_ref[0] = j_all[1:, :]


def _topk(point_cloud):
    return pl.pallas_call(
        _topk_body,
        grid=(_B, _N // _QB),
        in_specs=[
            pl.BlockSpec((1, _D, _QB), lambda b, q: (b, 0, q)),
            pl.BlockSpec((1, _D, _N), lambda b, q: (b, 0, 0)),
        ],
        out_specs=pl.BlockSpec((1, _K, _QB), lambda b, q: (b, 0, q)),
        out_shape=jax.ShapeDtypeStruct((_B, _K, _N), jnp.int32),
        scratch_shapes=[pltpu.VMEM((_QB, _N), jnp.float32)],
        compiler_params=pltpu.CompilerParams(
            dimension_semantics=("parallel", "parallel")),
    )(point_cloud, point_cloud)


def _edge_body(pc_hbm, idx_hbm, out_hbm, pcx_v, pcy_v, pcz_v, idx_v, out_v):
    wid = lax.axis_index("s") * 2 + lax.axis_index("c")
    b = wid // _TPB
    t = wid % _TPB
    n0 = pl.multiple_of(t * _CHUNK, _CHUNK)
    pltpu.sync_copy(pc_hbm.at[pl.ds((b * 3 + 0) * _N, _N)], pcx_v)
    pltpu.sync_copy(pc_hbm.at[pl.ds((b * 3 + 1) * _N, _N)], pcy_v)
    pltpu.sync_copy(pc_hbm.at[pl.ds((b * 3 + 2) * _N, _N)], pcz_v)
    for kk in range(_K):
        pltpu.sync_copy(idx_hbm.at[pl.ds((b * _K + kk) * _N + n0, _CHUNK)],
                        idx_v.at[pl.ds(kk * _CHUNK, _CHUNK)])

    def body(v, carry):
        off = pl.multiple_of(v * 16, 16)
        qoff = pl.multiple_of(n0 + off, 16)
        qx = pcx_v[pl.ds(qoff, 16)]
        qy = pcy_v[pl.ds(qoff, 16)]
        qz = pcz_v[pl.ds(qoff, 16)]
        for kk in range(_K):
            ii = idx_v[pl.ds(kk * _CHUNK + off, 16)]
            gx = plsc.load_gather(pcx_v, [ii])
            gy = plsc.load_gather(pcy_v, [ii])
            gz = plsc.load_gather(pcz_v, [ii])
            out_v[pl.ds((0 * _K + kk) * _CHUNK + off, 16)] = qx
            out_v[pl.ds((1 * _K + kk) * _CHUNK + off, 16)] = qy
            out_v[pl.ds((2 * _K + kk) * _CHUNK + off, 16)] = qz
            out_v[pl.ds((3 * _K + kk) * _CHUNK + off, 16)] = gx - qx
            out_v[pl.ds((4 * _K + kk) * _CHUNK + off, 16)] = gy - qy
            out_v[pl.ds((5 * _K + kk) * _CHUNK + off, 16)] = gz - qz
        return carry

    lax.fori_loop(0, _CHUNK // 16, body, 0)
    for c in range(2 * _D):
        for kk in range(_K):
            pltpu.sync_copy(
                out_v.at[pl.ds((c * _K + kk) * _CHUNK, _CHUNK)],
                out_hbm.at[pl.ds(((b * 2 * _D + c) * _K + kk) * _N + n0,
                                 _CHUNK)])


def _edge(point_cloud, idx):
    mesh = plsc.VectorSubcoreMesh(core_axis_name="c", subcore_axis_name="s")
    k = functools.partial(
        pl.kernel,
        mesh=mesh,
        out_type=jax.ShapeDtypeStruct((_B * 2 * _D * _K * _N,), jnp.float32),
        scratch_types=[
            pltpu.VMEM((_N,), jnp.float32),
            pltpu.VMEM((_N,), jnp.float32),
            pltpu.VMEM((_N,), jnp.float32),
            pltpu.VMEM((_K * _CHUNK,), jnp.int32),
            pltpu.VMEM((2 * _D * _K * _CHUNK,), jnp.float32),
        ],
        compiler_params=pltpu.CompilerParams(use_tc_tiling_on_sc=False,
                                             needs_layout_passes=False),
    )(_edge_body)
    out = k(point_cloud.reshape(-1), idx.reshape(-1))
    return out.reshape(_B, 2 * _D, _K, _N)


def kernel(point_cloud):
    idx = _topk(point_cloud)
    edge_feature = _edge(point_cloud, idx)
    return edge_feature, idx


# speedup vs baseline: 1.4217x; 1.0612x over previous
import functools

import jax
import jax.numpy as jnp
from jax import lax
from jax.experimental import pallas as pl
from jax.experimental.pallas import tpu as pltpu
from jax.experimental.pallas import tpu_sc as plsc

_K = 16
_B = 4
_D = 3
_N = 4096
_QB = 512
_NW = 32
_CHUNK = _B * _N // _NW
_TPB = _NW // _B


def _topk_body(pcq_ref, pcr_ref, idx_ref, vals_ref):
    qblk = pl.program_id(1)
    q = pcq_ref[0]
    r = pcr_ref[0]
    sq_q = jnp.sum(q * q, axis=0)
    sq_r = jnp.sum(r * r, axis=0)
    qb = q.astype(jnp.bfloat16).astype(jnp.float32)
    rb = r.astype(jnp.bfloat16).astype(jnp.float32)
    inner = (qb[0][:, None] * rb[0][None, :]
             + qb[1][:, None] * rb[1][None, :]
             + qb[2][:, None] * rb[2][None, :])
    d2 = (sq_r[None, :] + sq_q[:, None]) - 2.0 * inner

    lane = lax.broadcasted_iota(jnp.int32, (_QB, _N), 1)
    inf = jnp.float32(jnp.inf)
    vals_ref[...] = d2

    krow = lax.broadcasted_iota(jnp.int32, (_K + 1, _QB), 0)

    def round_body(kk, j_all):
        vals = vals_ref[...]
        m = jnp.min(vals, axis=1)
        cand = jnp.where(vals == m[:, None], lane, _N)
        j = jnp.min(cand, axis=1)
        j_all = jnp.where(krow == kk, j[None, :], j_all)
        vals_ref[...] = jnp.where(lane == j[:, None], inf, vals)
        return j_all

    j_all = lax.fori_loop(0, _K + 1, round_body,
                          jnp.zeros((_K + 1, _QB), jnp.int32))
    idx_ref[0] = j_all[1:, :]


def _topk(point_cloud):
    return pl.pallas_call(
        _topk_body,
        grid=(_B, _N // _QB),
        in_specs=[
            pl.BlockSpec((1, _D, _QB), lambda b, q: (b, 0, q)),
            pl.BlockSpec((1, _D, _N), lambda b, q: (b, 0, 0)),
        ],
        out_specs=pl.BlockSpec((1, _K, _QB), lambda b, q: (b, 0, q)),
        out_shape=jax.ShapeDtypeStruct((_B, _K, _N), jnp.int32),
        scratch_shapes=[pltpu.VMEM((_QB, _N), jnp.float32)],
        compiler_params=pltpu.CompilerParams(
            dimension_semantics=("parallel", "parallel")),
    )(point_cloud, point_cloud)


def _edge_body(pc_hbm, idx_hbm, out_hbm, pcx_v, pcy_v, pcz_v, idx_v, out_v):
    wid = lax.axis_index("s") * 2 + lax.axis_index("c")
    b = wid // _TPB
    t = wid % _TPB
    n0 = pl.multiple_of(t * _CHUNK, _CHUNK)
    pltpu.sync_copy(pc_hbm.at[pl.ds((b * 3 + 0) * _N, _N)], pcx_v)
    pltpu.sync_copy(pc_hbm.at[pl.ds((b * 3 + 1) * _N, _N)], pcy_v)
    pltpu.sync_copy(pc_hbm.at[pl.ds((b * 3 + 2) * _N, _N)], pcz_v)
    for kk in range(_K):
        pltpu.sync_copy(idx_hbm.at[pl.ds((b * _K + kk) * _N + n0, _CHUNK)],
                        idx_v.at[pl.ds(kk * _CHUNK, _CHUNK)])

    def body(v, carry):
        off = pl.multiple_of(v * 16, 16)
        qoff = pl.multiple_of(n0 + off, 16)
        qx = pcx_v[pl.ds(qoff, 16)]
        qy = pcy_v[pl.ds(qoff, 16)]
        qz = pcz_v[pl.ds(qoff, 16)]
        for kk in range(_K):
            ii = idx_v[pl.ds(kk * _CHUNK + off, 16)]
            gx = plsc.load_gather(pcx_v, [ii])
            gy = plsc.load_gather(pcy_v, [ii])
            gz = plsc.load_gather(pcz_v, [ii])
            out_v[pl.ds((0 * _K + kk) * _CHUNK + off, 16)] = qx
            out_v[pl.ds((1 * _K + kk) * _CHUNK + off, 16)] = qy
            out_v[pl.ds((2 * _K + kk) * _CHUNK + off, 16)] = qz
            out_v[pl.ds((3 * _K + kk) * _CHUNK + off, 16)] = gx - qx
            out_v[pl.ds((4 * _K + kk) * _CHUNK + off, 16)] = gy - qy
            out_v[pl.ds((5 * _K + kk) * _CHUNK + off, 16)] = gz - qz
        return carry

    lax.fori_loop(0, _CHUNK // 16, body, 0)
    for c in range(2 * _D):
        for kk in range(_K):
            pltpu.sync_copy(
                out_v.at[pl.ds((c * _K + kk) * _CHUNK, _CHUNK)],
                out_hbm.at[pl.ds(((b * 2 * _D + c) * _K + kk) * _N + n0,
                                 _CHUNK)])


def _edge(point_cloud, idx):
    mesh = plsc.VectorSubcoreMesh(core_axis_name="c", subcore_axis_name="s")
    k = functools.partial(
        pl.kernel,
        mesh=mesh,
        out_type=jax.ShapeDtypeStruct((_B * 2 * _D * _K * _N,), jnp.float32),
        scratch_types=[
            pltpu.VMEM((_N,), jnp.float32),
            pltpu.VMEM((_N,), jnp.float32),
            pltpu.VMEM((_N,), jnp.float32),
            pltpu.VMEM((_K * _CHUNK,), jnp.int32),
            pltpu.VMEM((2 * _D * _K * _CHUNK,), jnp.float32),
        ],
        compiler_params=pltpu.CompilerParams(use_tc_tiling_on_sc=False,
                                             needs_layout_passes=False),
    )(_edge_body)
    out = k(point_cloud.reshape(-1), idx.reshape(-1))
    return out.reshape(_B, 2 * _D, _K, _N)


def kernel(point_cloud):
    idx = _topk(point_cloud)
    edge_feature = _edge(point_cloud, idx)
    return edge_feature, idx


# speedup vs baseline: 1.4295x; 1.0055x over previous
import functools

import jax
import jax.numpy as jnp
from jax import lax
from jax.experimental import pallas as pl
from jax.experimental.pallas import tpu as pltpu
from jax.experimental.pallas import tpu_sc as plsc

_K = 16
_B = 4
_D = 3
_N = 4096
_QB = 1024
_NW = 32
_CHUNK = _B * _N // _NW
_TPB = _NW // _B


def _topk_body(pcq_ref, pcr_ref, idx_ref, vals_ref):
    qblk = pl.program_id(1)
    q = pcq_ref[0]
    r = pcr_ref[0]
    sq_q = jnp.sum(q * q, axis=0)
    sq_r = jnp.sum(r * r, axis=0)
    qb = q.astype(jnp.bfloat16).astype(jnp.float32)
    rb = r.astype(jnp.bfloat16).astype(jnp.float32)
    inner = (qb[0][:, None] * rb[0][None, :]
             + qb[1][:, None] * rb[1][None, :]
             + qb[2][:, None] * rb[2][None, :])
    d2 = (sq_r[None, :] + sq_q[:, None]) - 2.0 * inner

    lane = lax.broadcasted_iota(jnp.int32, (_QB, _N), 1)
    inf = jnp.float32(jnp.inf)
    vals_ref[...] = d2

    krow = lax.broadcasted_iota(jnp.int32, (_K + 1, _QB), 0)

    def round_body(kk, j_all):
        vals = vals_ref[...]
        m = jnp.min(vals, axis=1)
        cand = jnp.where(vals == m[:, None], lane, _N)
        j = jnp.min(cand, axis=1)
        j_all = jnp.where(krow == kk, j[None, :], j_all)
        vals_ref[...] = jnp.where(lane == j[:, None], inf, vals)
        return j_all

    j_all = lax.fori_loop(0, _K + 1, round_body,
                          jnp.zeros((_K + 1, _QB), jnp.int32))
    idx_ref[0] = j_all[1:, :]


def _topk(point_cloud):
    return pl.pallas_call(
        _topk_body,
        grid=(_B, _N // _QB),
        in_specs=[
            pl.BlockSpec((1, _D, _QB), lambda b, q: (b, 0, q)),
            pl.BlockSpec((1, _D, _N), lambda b, q: (b, 0, 0)),
        ],
        out_specs=pl.BlockSpec((1, _K, _QB), lambda b, q: (b, 0, q)),
        out_shape=jax.ShapeDtypeStruct((_B, _K, _N), jnp.int32),
        scratch_shapes=[pltpu.VMEM((_QB, _N), jnp.float32)],
        compiler_params=pltpu.CompilerParams(
            dimension_semantics=("parallel", "parallel")),
    )(point_cloud, point_cloud)


def _edge_body(pc_hbm, idx_hbm, out_hbm, pcx_v, pcy_v, pcz_v, idx_v, out_v):
    wid = lax.axis_index("s") * 2 + lax.axis_index("c")
    b = wid // _TPB
    t = wid % _TPB
    n0 = pl.multiple_of(t * _CHUNK, _CHUNK)
    pltpu.sync_copy(pc_hbm.at[pl.ds((b * 3 + 0) * _N, _N)], pcx_v)
    pltpu.sync_copy(pc_hbm.at[pl.ds((b * 3 + 1) * _N, _N)], pcy_v)
    pltpu.sync_copy(pc_hbm.at[pl.ds((b * 3 + 2) * _N, _N)], pcz_v)
    for kk in range(_K):
        pltpu.sync_copy(idx_hbm.at[pl.ds((b * _K + kk) * _N + n0, _CHUNK)],
                        idx_v.at[pl.ds(kk * _CHUNK, _CHUNK)])

    def body(v, carry):
        off = pl.multiple_of(v * 16, 16)
        qoff = pl.multiple_of(n0 + off, 16)
        qx = pcx_v[pl.ds(qoff, 16)]
        qy = pcy_v[pl.ds(qoff, 16)]
        qz = pcz_v[pl.ds(qoff, 16)]
        for kk in range(_K):
            ii = idx_v[pl.ds(kk * _CHUNK + off, 16)]
            gx = plsc.load_gather(pcx_v, [ii])
            gy = plsc.load_gather(pcy_v, [ii])
            gz = plsc.load_gather(pcz_v, [ii])
            out_v[pl.ds((0 * _K + kk) * _CHUNK + off, 16)] = qx
            out_v[pl.ds((1 * _K + kk) * _CHUNK + off, 16)] = qy
            out_v[pl.ds((2 * _K + kk) * _CHUNK + off, 16)] = qz
            out_v[pl.ds((3 * _K + kk) * _CHUNK + off, 16)] = gx - qx
            out_v[pl.ds((4 * _K + kk) * _CHUNK + off, 16)] = gy - qy
            out_v[pl.ds((5 * _K + kk) * _CHUNK + off, 16)] = gz - qz
        return carry

    lax.fori_loop(0, _CHUNK // 16, body, 0)
    for c in range(2 * _D):
        for kk in range(_K):
            pltpu.sync_copy(
                out_v.at[pl.ds((c * _K + kk) * _CHUNK, _CHUNK)],
                out_hbm.at[pl.ds(((b * 2 * _D + c) * _K + kk) * _N + n0,
                                 _CHUNK)])


def _edge(point_cloud, idx):
    mesh = plsc.VectorSubcoreMesh(core_axis_name="c", subcore_axis_name="s")
    k = functools.partial(
        pl.kernel,
        mesh=mesh,
        out_type=jax.ShapeDtypeStruct((_B * 2 * _D * _K * _N,), jnp.float32),
        scratch_types=[
            pltpu.VMEM((_N,), jnp.float32),
            pltpu.VMEM((_N,), jnp.float32),
            pltpu.VMEM((_N,), jnp.float32),
            pltpu.VMEM((_K * _CHUNK,), jnp.int32),
            pltpu.VMEM((2 * _D * _K * _CHUNK,), jnp.float32),
        ],
        compiler_params=pltpu.CompilerParams(use_tc_tiling_on_sc=False,
                                             needs_layout_passes=False),
    )(_edge_body)
    out = k(point_cloud.reshape(-1), idx.reshape(-1))
    return out.reshape(_B, 2 * _D, _K, _N)


def kernel(point_cloud):
    idx = _topk(point_cloud)
    edge_feature = _edge(point_cloud, idx)
    return edge_feature, idx


# speedup vs baseline: 1.6677x; 1.1666x over previous
import functools

import jax
import jax.numpy as jnp
from jax import lax
from jax.experimental import pallas as pl
from jax.experimental.pallas import tpu as pltpu
from jax.experimental.pallas import tpu_sc as plsc

_K = 16
_B = 4
_D = 3
_N = 4096
_QB = 1024
_NW = 32
_CHUNK = _B * _N // _NW
_TPB = _NW // _B


def _topk_body(pcq_ref, pcr_ref, idx_ref, vals_ref):
    qblk = pl.program_id(1)
    q = pcq_ref[0]
    r = pcr_ref[0]
    sq_q = jnp.sum(q * q, axis=0)
    sq_r = jnp.sum(r * r, axis=0)
    qb = q.astype(jnp.bfloat16).astype(jnp.float32)
    rb = r.astype(jnp.bfloat16).astype(jnp.float32)
    inner = (qb[0][:, None] * rb[0][None, :]
             + qb[1][:, None] * rb[1][None, :]
             + qb[2][:, None] * rb[2][None, :])
    d2 = (sq_r[None, :] + sq_q[:, None]) - 2.0 * inner

    lane = lax.broadcasted_iota(jnp.int32, (_QB, _N), 1).astype(jnp.float32)
    inf = jnp.float32(jnp.inf)
    big = jnp.float32(_N)
    vals_ref[...] = d2

    krow = lax.broadcasted_iota(jnp.int32, (_K + 1, _QB), 0)

    def round_body(kk, j_all):
        vals = vals_ref[...]
        m = jnp.min(vals, axis=1)
        cand = jnp.where(vals == m[:, None], lane, big)
        j = jnp.min(cand, axis=1)
        j_all = jnp.where(krow == kk, j[None, :], j_all)
        vals_ref[...] = jnp.where(lane == j[:, None], inf, vals)
        return j_all

    j_all = lax.fori_loop(0, _K + 1, round_body,
                          jnp.zeros((_K + 1, _QB), jnp.float32))
    idx_ref[0] = j_all[1:, :].astype(jnp.int32)


def _topk(point_cloud):
    return pl.pallas_call(
        _topk_body,
        grid=(_B, _N // _QB),
        in_specs=[
            pl.BlockSpec((1, _D, _QB), lambda b, q: (b, 0, q)),
            pl.BlockSpec((1, _D, _N), lambda b, q: (b, 0, 0)),
        ],
        out_specs=pl.BlockSpec((1, _K, _QB), lambda b, q: (b, 0, q)),
        out_shape=jax.ShapeDtypeStruct((_B, _K, _N), jnp.int32),
        scratch_shapes=[pltpu.VMEM((_QB, _N), jnp.float32)],
        compiler_params=pltpu.CompilerParams(
            dimension_semantics=("parallel", "parallel")),
    )(point_cloud, point_cloud)


def _edge_body(pc_hbm, idx_hbm, out_hbm, pcx_v, pcy_v, pcz_v, idx_v, out_v):
    wid = lax.axis_index("s") * 2 + lax.axis_index("c")
    b = wid // _TPB
    t = wid % _TPB
    n0 = pl.multiple_of(t * _CHUNK, _CHUNK)
    pltpu.sync_copy(pc_hbm.at[pl.ds((b * 3 + 0) * _N, _N)], pcx_v)
    pltpu.sync_copy(pc_hbm.at[pl.ds((b * 3 + 1) * _N, _N)], pcy_v)
    pltpu.sync_copy(pc_hbm.at[pl.ds((b * 3 + 2) * _N, _N)], pcz_v)
    for kk in range(_K):
        pltpu.sync_copy(idx_hbm.at[pl.ds((b * _K + kk) * _N + n0, _CHUNK)],
                        idx_v.at[pl.ds(kk * _CHUNK, _CHUNK)])

    def body(v, carry):
        off = pl.multiple_of(v * 16, 16)
        qoff = pl.multiple_of(n0 + off, 16)
        qx = pcx_v[pl.ds(qoff, 16)]
        qy = pcy_v[pl.ds(qoff, 16)]
        qz = pcz_v[pl.ds(qoff, 16)]
        for kk in range(_K):
            ii = idx_v[pl.ds(kk * _CHUNK + off, 16)]
            gx = plsc.load_gather(pcx_v, [ii])
            gy = plsc.load_gather(pcy_v, [ii])
            gz = plsc.load_gather(pcz_v, [ii])
            out_v[pl.ds((0 * _K + kk) * _CHUNK + off, 16)] = qx
            out_v[pl.ds((1 * _K + kk) * _CHUNK + off, 16)] = qy
            out_v[pl.ds((2 * _K + kk) * _CHUNK + off, 16)] = qz
            out_v[pl.ds((3 * _K + kk) * _CHUNK + off, 16)] = gx - qx
            out_v[pl.ds((4 * _K + kk) * _CHUNK + off, 16)] = gy - qy
            out_v[pl.ds((5 * _K + kk) * _CHUNK + off, 16)] = gz - qz
        return carry

    lax.fori_loop(0, _CHUNK // 16, body, 0)
    for c in range(2 * _D):
        for kk in range(_K):
            pltpu.sync_copy(
                out_v.at[pl.ds((c * _K + kk) * _CHUNK, _CHUNK)],
                out_hbm.at[pl.ds(((b * 2 * _D + c) * _K + kk) * _N + n0,
                                 _CHUNK)])


def _edge(point_cloud, idx):
    mesh = plsc.VectorSubcoreMesh(core_axis_name="c", subcore_axis_name="s")
    k = functools.partial(
        pl.kernel,
        mesh=mesh,
        out_type=jax.ShapeDtypeStruct((_B * 2 * _D * _K * _N,), jnp.float32),
        scratch_types=[
            pltpu.VMEM((_N,), jnp.float32),
            pltpu.VMEM((_N,), jnp.float32),
            pltpu.VMEM((_N,), jnp.float32),
            pltpu.VMEM((_K * _CHUNK,), jnp.int32),
            pltpu.VMEM((2 * _D * _K * _CHUNK,), jnp.float32),
        ],
        compiler_params=pltpu.CompilerParams(use_tc_tiling_on_sc=False,
                                             needs_layout_passes=False),
    )(_edge_body)
    out = k(point_cloud.reshape(-1), idx.reshape(-1))
    return out.reshape(_B, 2 * _D, _K, _N)


def kernel(point_cloud):
    idx = _topk(point_cloud)
    edge_feature = _edge(point_cloud, idx)
    return edge_feature, idx


# speedup vs baseline: 1.7580x; 1.0542x over previous
import functools

import jax
import jax.numpy as jnp
from jax import lax
from jax.experimental import pallas as pl
from jax.experimental.pallas import tpu as pltpu
from jax.experimental.pallas import tpu_sc as plsc

_K = 16
_B = 4
_D = 3
_N = 4096
_QB = 1024
_NW = 32
_CHUNK = _B * _N // _NW
_TPB = _NW // _B


def _topk_body(pcq_ref, pcr_ref, idx_ref, vals_ref):
    qblk = pl.program_id(1)
    q = pcq_ref[0]
    r = pcr_ref[0]
    sq_q = jnp.sum(q * q, axis=0)
    sq_r = jnp.sum(r * r, axis=0)
    qm2 = (-2.0 * q).astype(jnp.bfloat16)
    rb = r.astype(jnp.bfloat16)
    inner2 = lax.dot_general(qm2, rb, (((0,), (0,)), ((), ())),
                             preferred_element_type=jnp.float32)
    d2 = (sq_r[None, :] + sq_q[:, None]) + inner2

    lane = lax.broadcasted_iota(jnp.int32, (_QB, _N), 1).astype(jnp.float32)
    inf = jnp.float32(jnp.inf)
    big = jnp.float32(_N)
    vals_ref[...] = d2

    krow = lax.broadcasted_iota(jnp.int32, (_K + 1, _QB), 0)

    def round_body(kk, j_all):
        vals = vals_ref[...]
        m = jnp.min(vals, axis=1)
        cand = jnp.where(vals == m[:, None], lane, big)
        j = jnp.min(cand, axis=1)
        j_all = jnp.where(krow == kk, j[None, :], j_all)
        vals_ref[...] = jnp.where(lane == j[:, None], inf, vals)
        return j_all

    j_all = lax.fori_loop(0, _K + 1, round_body,
                          jnp.zeros((_K + 1, _QB), jnp.float32))
    idx_ref[0] = j_all[1:, :].astype(jnp.int32)


def _topk(point_cloud):
    return pl.pallas_call(
        _topk_body,
        grid=(_B, _N // _QB),
        in_specs=[
            pl.BlockSpec((1, _D, _QB), lambda b, q: (b, 0, q)),
            pl.BlockSpec((1, _D, _N), lambda b, q: (b, 0, 0)),
        ],
        out_specs=pl.BlockSpec((1, _K, _QB), lambda b, q: (b, 0, q)),
        out_shape=jax.ShapeDtypeStruct((_B, _K, _N), jnp.int32),
        scratch_shapes=[pltpu.VMEM((_QB, _N), jnp.float32)],
        compiler_params=pltpu.CompilerParams(
            dimension_semantics=("parallel", "parallel")),
    )(point_cloud, point_cloud)


def _edge_body(pc_hbm, idx_hbm, out_hbm, pcx_v, pcy_v, pcz_v, idx_v, out_v):
    wid = lax.axis_index("s") * 2 + lax.axis_index("c")
    b = wid // _TPB
    t = wid % _TPB
    n0 = pl.multiple_of(t * _CHUNK, _CHUNK)
    pltpu.sync_copy(pc_hbm.at[pl.ds((b * 3 + 0) * _N, _N)], pcx_v)
    pltpu.sync_copy(pc_hbm.at[pl.ds((b * 3 + 1) * _N, _N)], pcy_v)
    pltpu.sync_copy(pc_hbm.at[pl.ds((b * 3 + 2) * _N, _N)], pcz_v)
    for kk in range(_K):
        pltpu.sync_copy(idx_hbm.at[pl.ds((b * _K + kk) * _N + n0, _CHUNK)],
                        idx_v.at[pl.ds(kk * _CHUNK, _CHUNK)])

    def body(v, carry):
        off = pl.multiple_of(v * 16, 16)
        qoff = pl.multiple_of(n0 + off, 16)
        qx = pcx_v[pl.ds(qoff, 16)]
        qy = pcy_v[pl.ds(qoff, 16)]
        qz = pcz_v[pl.ds(qoff, 16)]
        for kk in range(_K):
            ii = idx_v[pl.ds(kk * _CHUNK + off, 16)]
            gx = plsc.load_gather(pcx_v, [ii])
            gy = plsc.load_gather(pcy_v, [ii])
            gz = plsc.load_gather(pcz_v, [ii])
            out_v[pl.ds((0 * _K + kk) * _CHUNK + off, 16)] = qx
            out_v[pl.ds((1 * _K + kk) * _CHUNK + off, 16)] = qy
            out_v[pl.ds((2 * _K + kk) * _CHUNK + off, 16)] = qz
            out_v[pl.ds((3 * _K + kk) * _CHUNK + off, 16)] = gx - qx
            out_v[pl.ds((4 * _K + kk) * _CHUNK + off, 16)] = gy - qy
            out_v[pl.ds((5 * _K + kk) * _CHUNK + off, 16)] = gz - qz
        return carry

    lax.fori_loop(0, _CHUNK // 16, body, 0)
    for c in range(2 * _D):
        for kk in range(_K):
            pltpu.sync_copy(
                out_v.at[pl.ds((c * _K + kk) * _CHUNK, _CHUNK)],
                out_hbm.at[pl.ds(((b * 2 * _D + c) * _K + kk) * _N + n0,
                                 _CHUNK)])


def _edge(point_cloud, idx):
    mesh = plsc.VectorSubcoreMesh(core_axis_name="c", subcore_axis_name="s")
    k = functools.partial(
        pl.kernel,
        mesh=mesh,
        out_type=jax.ShapeDtypeStruct((_B * 2 * _D * _K * _N,), jnp.float32),
        scratch_types=[
            pltpu.VMEM((_N,), jnp.float32),
            pltpu.VMEM((_N,), jnp.float32),
            pltpu.VMEM((_N,), jnp.float32),
            pltpu.VMEM((_K * _CHUNK,), jnp.int32),
            pltpu.VMEM((2 * _D * _K * _CHUNK,), jnp.float32),
        ],
        compiler_params=pltpu.CompilerParams(use_tc_tiling_on_sc=False,
                                             needs_layout_passes=False),
    )(_edge_body)
    out = k(point_cloud.reshape(-1), idx.reshape(-1))
    return out.reshape(_B, 2 * _D, _K, _N)


def kernel(point_cloud):
    idx = _topk(point_cloud)
    edge_feature = _edge(point_cloud, idx)
    return edge_feature, idx


# speedup vs baseline: 1.7751x; 1.0097x over previous
import functools

import jax
import jax.numpy as jnp
from jax import lax
from jax.experimental import pallas as pl
from jax.experimental.pallas import tpu as pltpu
from jax.experimental.pallas import tpu_sc as plsc

_K = 16
_B = 4
_D = 3
_N = 4096
_QB = 2048
_NW = 32
_CHUNK = _B * _N // _NW
_TPB = _NW // _B


def _topk_body(pcq_ref, pcr_ref, idx_ref, vals_ref):
    qblk = pl.program_id(1)
    q = pcq_ref[0]
    r = pcr_ref[0]
    sq_q = jnp.sum(q * q, axis=0)
    sq_r = jnp.sum(r * r, axis=0)
    qm2 = (-2.0 * q).astype(jnp.bfloat16)
    rb = r.astype(jnp.bfloat16)
    inner2 = lax.dot_general(qm2, rb, (((0,), (0,)), ((), ())),
                             preferred_element_type=jnp.float32)
    d2 = (sq_r[None, :] + sq_q[:, None]) + inner2

    lane = lax.broadcasted_iota(jnp.int32, (_QB, _N), 1).astype(jnp.float32)
    inf = jnp.float32(jnp.inf)
    big = jnp.float32(_N)
    vals_ref[...] = d2

    krow = lax.broadcasted_iota(jnp.int32, (_K + 1, _QB), 0)

    def round_body(kk, j_all):
        vals = vals_ref[...]
        m = jnp.min(vals, axis=1)
        cand = jnp.where(vals == m[:, None], lane, big)
        j = jnp.min(cand, axis=1)
        j_all = jnp.where(krow == kk, j[None, :], j_all)
        vals_ref[...] = jnp.where(lane == j[:, None], inf, vals)
        return j_all

    j_all = lax.fori_loop(0, _K + 1, round_body,
                          jnp.zeros((_K + 1, _QB), jnp.float32))
    idx_ref[0] = j_all[1:, :].astype(jnp.int32)


def _topk(point_cloud):
    return pl.pallas_call(
        _topk_body,
        grid=(_B, _N // _QB),
        in_specs=[
            pl.BlockSpec((1, _D, _QB), lambda b, q: (b, 0, q)),
            pl.BlockSpec((1, _D, _N), lambda b, q: (b, 0, 0)),
        ],
        out_specs=pl.BlockSpec((1, _K, _QB), lambda b, q: (b, 0, q)),
        out_shape=jax.ShapeDtypeStruct((_B, _K, _N), jnp.int32),
        scratch_shapes=[pltpu.VMEM((_QB, _N), jnp.float32)],
        compiler_params=pltpu.CompilerParams(
            dimension_semantics=("parallel", "parallel")),
    )(point_cloud, point_cloud)


def _edge_body(pc_hbm, idx_hbm, out_hbm, pcx_v, pcy_v, pcz_v, idx_v, out_v):
    wid = lax.axis_index("s") * 2 + lax.axis_index("c")
    b = wid // _TPB
    t = wid % _TPB
    n0 = pl.multiple_of(t * _CHUNK, _CHUNK)
    pltpu.sync_copy(pc_hbm.at[pl.ds((b * 3 + 0) * _N, _N)], pcx_v)
    pltpu.sync_copy(pc_hbm.at[pl.ds((b * 3 + 1) * _N, _N)], pcy_v)
    pltpu.sync_copy(pc_hbm.at[pl.ds((b * 3 + 2) * _N, _N)], pcz_v)
    for kk in range(_K):
        pltpu.sync_copy(idx_hbm.at[pl.ds((b * _K + kk) * _N + n0, _CHUNK)],
                        idx_v.at[pl.ds(kk * _CHUNK, _CHUNK)])

    def body(v, carry):
        off = pl.multiple_of(v * 16, 16)
        qoff = pl.multiple_of(n0 + off, 16)
        qx = pcx_v[pl.ds(qoff, 16)]
        qy = pcy_v[pl.ds(qoff, 16)]
        qz = pcz_v[pl.ds(qoff, 16)]
        for kk in range(_K):
            ii = idx_v[pl.ds(kk * _CHUNK + off, 16)]
            gx = plsc.load_gather(pcx_v, [ii])
            gy = plsc.load_gather(pcy_v, [ii])
            gz = plsc.load_gather(pcz_v, [ii])
            out_v[pl.ds((0 * _K + kk) * _CHUNK + off, 16)] = qx
            out_v[pl.ds((1 * _K + kk) * _CHUNK + off, 16)] = qy
            out_v[pl.ds((2 * _K + kk) * _CHUNK + off, 16)] = qz
            out_v[pl.ds((3 * _K + kk) * _CHUNK + off, 16)] = gx - qx
            out_v[pl.ds((4 * _K + kk) * _CHUNK + off, 16)] = gy - qy
            out_v[pl.ds((5 * _K + kk) * _CHUNK + off, 16)] = gz - qz
        return carry

    lax.fori_loop(0, _CHUNK // 16, body, 0)
    for c in range(2 * _D):
        for kk in range(_K):
            pltpu.sync_copy(
                out_v.at[pl.ds((c * _K + kk) * _CHUNK, _CHUNK)],
                out_hbm.at[pl.ds(((b * 2 * _D + c) * _K + kk) * _N + n0,
                                 _CHUNK)])


def _edge(point_cloud, idx):
    mesh = plsc.VectorSubcoreMesh(core_axis_name="c", subcore_axis_name="s")
    k = functools.partial(
        pl.kernel,
        mesh=mesh,
        out_type=jax.ShapeDtypeStruct((_B * 2 * _D * _K * _N,), jnp.float32),
        scratch_types=[
            pltpu.VMEM((_N,), jnp.float32),
            pltpu.VMEM((_N,), jnp.float32),
            pltpu.VMEM((_N,), jnp.float32),
            pltpu.VMEM((_K * _CHUNK,), jnp.int32),
            pltpu.VMEM((2 * _D * _K * _CHUNK,), jnp.float32),
        ],
        compiler_params=pltpu.CompilerParams(use_tc_tiling_on_sc=False,
                                             needs_layout_passes=False),
    )(_edge_body)
    out = k(point_cloud.reshape(-1), idx.reshape(-1))
    return out.reshape(_B, 2 * _D, _K, _N)


def kernel(point_cloud):
    idx = _topk(point_cloud)
    edge_feature = _edge(point_cloud, idx)
    return edge_feature, idx


# speedup vs baseline: 1.8179x; 1.0241x over previous
import functools

import jax
import jax.numpy as jnp
from jax import lax
from jax.experimental import pallas as pl
from jax.experimental.pallas import tpu as pltpu
from jax.experimental.pallas import tpu_sc as plsc

_K = 16
_B = 4
_D = 3
_N = 4096
_QB = 2048
_NW = 32
_CHUNK = _B * _N // _NW
_TPB = _NW // _B


def _topk_body(pcq_ref, pcr_ref, idx_ref, vals_ref):
    qblk = pl.program_id(1)
    q = pcq_ref[0]
    r = pcr_ref[0]
    sq_q = jnp.sum(q * q, axis=0)
    sq_r = jnp.sum(r * r, axis=0)
    qm2 = (-2.0 * q).astype(jnp.bfloat16)
    rb = r.astype(jnp.bfloat16)
    inner2 = lax.dot_general(qm2, rb, (((0,), (0,)), ((), ())),
                             preferred_element_type=jnp.float32)
    d2 = (sq_r[None, :] + sq_q[:, None]) + inner2

    lane = lax.broadcasted_iota(jnp.int32, (_QB, _N), 1).astype(jnp.float32)
    inf = jnp.float32(jnp.inf)
    big = jnp.float32(_N)
    vals_ref[...] = d2

    krow = lax.broadcasted_iota(jnp.int32, (_K + 1, _QB), 0)

    def round_body(kk, j_all):
        vals = vals_ref[...]
        m = jnp.min(vals, axis=1)
        cand = jnp.where(vals == m[:, None], lane, big)
        j = jnp.min(cand, axis=1)
        j_all = jnp.where(krow == kk, j[None, :], j_all)
        vals_ref[...] = jnp.where(lane == j[:, None], inf, vals)
        return j_all

    j_all = lax.fori_loop(0, _K, round_body,
                          jnp.zeros((_K + 1, _QB), jnp.float32))
    vals = vals_ref[...]
    m = jnp.min(vals, axis=1)
    j = jnp.min(jnp.where(vals == m[:, None], lane, big), axis=1)
    j_all = jnp.where(krow == _K, j[None, :], j_all)
    idx_ref[0] = j_all[1:, :].astype(jnp.int32)


def _topk(point_cloud):
    return pl.pallas_call(
        _topk_body,
        grid=(_B, _N // _QB),
        in_specs=[
            pl.BlockSpec((1, _D, _QB), lambda b, q: (b, 0, q)),
            pl.BlockSpec((1, _D, _N), lambda b, q: (b, 0, 0)),
        ],
        out_specs=pl.BlockSpec((1, _K, _QB), lambda b, q: (b, 0, q)),
        out_shape=jax.ShapeDtypeStruct((_B, _K, _N), jnp.int32),
        scratch_shapes=[pltpu.VMEM((_QB, _N), jnp.float32)],
        compiler_params=pltpu.CompilerParams(
            dimension_semantics=("parallel", "parallel")),
    )(point_cloud, point_cloud)


def _edge_body(pc_hbm, idx_hbm, out_hbm, pcx_v, pcy_v, pcz_v, idx_v, out_v):
    wid = lax.axis_index("s") * 2 + lax.axis_index("c")
    b = wid // _TPB
    t = wid % _TPB
    n0 = pl.multiple_of(t * _CHUNK, _CHUNK)
    pltpu.sync_copy(pc_hbm.at[pl.ds((b * 3 + 0) * _N, _N)], pcx_v)
    pltpu.sync_copy(pc_hbm.at[pl.ds((b * 3 + 1) * _N, _N)], pcy_v)
    pltpu.sync_copy(pc_hbm.at[pl.ds((b * 3 + 2) * _N, _N)], pcz_v)
    for kk in range(_K):
        pltpu.sync_copy(idx_hbm.at[pl.ds((b * _K + kk) * _N + n0, _CHUNK)],
                        idx_v.at[pl.ds(kk * _CHUNK, _CHUNK)])

    def body(v, carry):
        off = pl.multiple_of(v * 16, 16)
        qoff = pl.multiple_of(n0 + off, 16)
        qx = pcx_v[pl.ds(qoff, 16)]
        qy = pcy_v[pl.ds(qoff, 16)]
        qz = pcz_v[pl.ds(qoff, 16)]
        for kk in range(_K):
            ii = idx_v[pl.ds(kk * _CHUNK + off, 16)]
            gx = plsc.load_gather(pcx_v, [ii])
            gy = plsc.load_gather(pcy_v, [ii])
            gz = plsc.load_gather(pcz_v, [ii])
            out_v[pl.ds((0 * _K + kk) * _CHUNK + off, 16)] = qx
            out_v[pl.ds((1 * _K + kk) * _CHUNK + off, 16)] = qy
            out_v[pl.ds((2 * _K + kk) * _CHUNK + off, 16)] = qz
            out_v[pl.ds((3 * _K + kk) * _CHUNK + off, 16)] = gx - qx
            out_v[pl.ds((4 * _K + kk) * _CHUNK + off, 16)] = gy - qy
            out_v[pl.ds((5 * _K + kk) * _CHUNK + off, 16)] = gz - qz
        return carry

    lax.fori_loop(0, _CHUNK // 16, body, 0)
    for c in range(2 * _D):
        for kk in range(_K):
            pltpu.sync_copy(
                out_v.at[pl.ds((c * _K + kk) * _CHUNK, _CHUNK)],
                out_hbm.at[pl.ds(((b * 2 * _D + c) * _K + kk) * _N + n0,
                                 _CHUNK)])


def _edge(point_cloud, idx):
    mesh = plsc.VectorSubcoreMesh(core_axis_name="c", subcore_axis_name="s")
    k = functools.partial(
        pl.kernel,
        mesh=mesh,
        out_type=jax.ShapeDtypeStruct((_B * 2 * _D * _K * _N,), jnp.float32),
        scratch_types=[
            pltpu.VMEM((_N,), jnp.float32),
            pltpu.VMEM((_N,), jnp.float32),
            pltpu.VMEM((_N,), jnp.float32),
            pltpu.VMEM((_K * _CHUNK,), jnp.int32),
            pltpu.VMEM((2 * _D * _K * _CHUNK,), jnp.float32),
        ],
        compiler_params=pltpu.CompilerParams(use_tc_tiling_on_sc=False,
                                             needs_layout_passes=False),
    )(_edge_body)
    out = k(point_cloud.reshape(-1), idx.reshape(-1))
    return out.reshape(_B, 2 * _D, _K, _N)


def kernel(point_cloud):
    idx = _topk(point_cloud)
    edge_feature = _edge(point_cloud, idx)
    return edge_feature, idx


# speedup vs baseline: 1.8184x; 1.0003x over previous
import functools

import jax
import jax.numpy as jnp
from jax import lax
from jax.experimental import pallas as pl
from jax.experimental.pallas import tpu as pltpu
from jax.experimental.pallas import tpu_sc as plsc

_K = 16
_B = 4
_D = 3
_N = 4096
_QB = 2048
_NW = 32
_CHUNK = _B * _N // _NW
_TPB = _NW // _B


def _topk_body(pcq_ref, pcr_ref, idx_ref, vals_ref):
    q = pcq_ref[0]
    r = pcr_ref[0]
    sq_q = jnp.sum(q * q, axis=0)
    sq_r = jnp.sum(r * r, axis=0)
    qm2 = (-2.0 * q).astype(jnp.bfloat16)
    rb = r.astype(jnp.bfloat16)
    inner2 = lax.dot_general(qm2, rb, (((0,), (0,)), ((), ())),
                             preferred_element_type=jnp.float32)
    d2 = (sq_r[None, :] + sq_q[:, None]) + inner2

    lane = lax.broadcasted_iota(jnp.int32, (_QB, _N), 1).astype(jnp.float32)
    inf = jnp.float32(jnp.inf)
    big = jnp.float32(_N)
    vals_ref[...] = d2

    krow = lax.broadcasted_iota(jnp.int32, (_K + 1, _QB), 0)

    def round_body(kk, j_all):
        vals = vals_ref[...]
        m = jnp.min(vals, axis=1)
        cand = jnp.where(vals == m[:, None], lane, big)
        j = jnp.min(cand, axis=1)
        j_all = jnp.where(krow == kk, j[None, :], j_all)
        vals_ref[...] = jnp.where(lane == j[:, None], inf, vals)
        return j_all

    j_all = lax.fori_loop(0, _K, round_body,
                          jnp.zeros((_K + 1, _QB), jnp.float32))
    vals = vals_ref[...]
    m = jnp.min(vals, axis=1)
    j = jnp.min(jnp.where(vals == m[:, None], lane, big), axis=1)
    j_all = jnp.where(krow == _K, j[None, :], j_all)
    idx_ref[0] = j_all[1:, :].astype(jnp.int32)


def _topk(point_cloud):
    return pl.pallas_call(
        _topk_body,
        grid=(_B, _N // _QB),
        in_specs=[
            pl.BlockSpec((1, _D, _QB), lambda b, q: (b, 0, q)),
            pl.BlockSpec((1, _D, _N), lambda b, q: (b, 0, 0)),
        ],
        out_specs=pl.BlockSpec((1, _K, _QB), lambda b, q: (b, 0, q)),
        out_shape=jax.ShapeDtypeStruct((_B, _K, _N), jnp.int32),
        scratch_shapes=[pltpu.VMEM((_QB, _N), jnp.float32)],
        compiler_params=pltpu.CompilerParams(
            dimension_semantics=("parallel", "parallel")),
    )(point_cloud, point_cloud)


def _edge_body(pc_hbm, idx_hbm, out_hbm, pcx_v, pcy_v, pcz_v, idx_v, out_v):
    wid = lax.axis_index("s") * 2 + lax.axis_index("c")
    b = wid // _TPB
    t = wid % _TPB
    n0 = pl.multiple_of(t * _CHUNK, _CHUNK)
    pltpu.sync_copy(pc_hbm.at[pl.ds((b * 3 + 0) * _N, _N)], pcx_v)
    pltpu.sync_copy(pc_hbm.at[pl.ds((b * 3 + 1) * _N, _N)], pcy_v)
    pltpu.sync_copy(pc_hbm.at[pl.ds((b * 3 + 2) * _N, _N)], pcz_v)
    for kk in range(_K):
        pltpu.sync_copy(idx_hbm.at[pl.ds((b * _K + kk) * _N + n0, _CHUNK)],
                        idx_v.at[pl.ds(kk * _CHUNK, _CHUNK)])

    def body(v, carry):
        off = pl.multiple_of(v * 16, 16)
        qoff = pl.multiple_of(n0 + off, 16)
        qx = pcx_v[pl.ds(qoff, 16)]
        qy = pcy_v[pl.ds(qoff, 16)]
        qz = pcz_v[pl.ds(qoff, 16)]
        for kk in range(_K):
            ii = idx_v[pl.ds(kk * _CHUNK + off, 16)]
            gx = plsc.load_gather(pcx_v, [ii])
            gy = plsc.load_gather(pcy_v, [ii])
            gz = plsc.load_gather(pcz_v, [ii])
            out_v[pl.ds((0 * _K + kk) * _CHUNK + off, 16)] = qx
            out_v[pl.ds((1 * _K + kk) * _CHUNK + off, 16)] = qy
            out_v[pl.ds((2 * _K + kk) * _CHUNK + off, 16)] = qz
            out_v[pl.ds((3 * _K + kk) * _CHUNK + off, 16)] = gx - qx
            out_v[pl.ds((4 * _K + kk) * _CHUNK + off, 16)] = gy - qy
            out_v[pl.ds((5 * _K + kk) * _CHUNK + off, 16)] = gz - qz
        return carry

    lax.fori_loop(0, _CHUNK // 16, body, 0)
    for c in range(2 * _D):
        for kk in range(_K):
            pltpu.sync_copy(
                out_v.at[pl.ds((c * _K + kk) * _CHUNK, _CHUNK)],
                out_hbm.at[pl.ds(((b * 2 * _D + c) * _K + kk) * _N + n0,
                                 _CHUNK)])


def _edge(point_cloud, idx):
    mesh = plsc.VectorSubcoreMesh(core_axis_name="c", subcore_axis_name="s")
    k = functools.partial(
        pl.kernel,
        mesh=mesh,
        out_type=jax.ShapeDtypeStruct((_B * 2 * _D * _K * _N,), jnp.float32),
        scratch_types=[
            pltpu.VMEM((_N,), jnp.float32),
            pltpu.VMEM((_N,), jnp.float32),
            pltpu.VMEM((_N,), jnp.float32),
            pltpu.VMEM((_K * _CHUNK,), jnp.int32),
            pltpu.VMEM((2 * _D * _K * _CHUNK,), jnp.float32),
        ],
        compiler_params=pltpu.CompilerParams(use_tc_tiling_on_sc=False,
                                             needs_layout_passes=False),
    )(_edge_body)
    out = k(point_cloud.reshape(-1), idx.reshape(-1))
    return out.reshape(_B, 2 * _D, _K, _N)


def kernel(point_cloud):
    idx = _topk(point_cloud)
    edge_feature = _edge(point_cloud, idx)
    return edge_feature, idx
